# baseline (speedup 1.0000x reference)
import functools

import jax
import jax.numpy as jnp
from jax import lax
from jax.experimental import pallas as pl
from jax.experimental.pallas import tpu as pltpu
from jax.experimental.pallas import tpu_sc as plsc

N = 4096
C = 1024
E = 8
F = 4096
TS = F // E
CHUNK = 128
NW = N // CHUNK
BM = 256
NT = N // BM
NSLOT = NT + E - 1


def _gate_body(x_ref, wg_ref, bg_ref, gate_ref, rank_ref, cnt_ref):
    xb = x_ref[...]
    logits = jnp.dot(xb, wg_ref[...],
                     preferred_element_type=jnp.float32) + bg_ref[...]
    lane = lax.broadcasted_iota(jnp.int32, logits.shape, 1)
    mx = jnp.max(logits, axis=-1, keepdims=True)
    idx = jnp.min(jnp.where(logits == mx, lane, E), axis=-1,
                  keepdims=True)
    onehot = (lane == idx).astype(jnp.float32)
    gate_ref[...] = onehot
    ri = lax.broadcasted_iota(jnp.int32, (CHUNK, CHUNK), 0)
    ci = lax.broadcasted_iota(jnp.int32, (CHUNK, CHUNK), 1)
    tril = (ci <= ri).astype(jnp.float32)
    csum = jnp.dot(tril, onehot, preferred_element_type=jnp.float32)
    rank = jnp.sum(onehot * csum, axis=-1, keepdims=True) - 1.0
    rank_ref[...] = rank.astype(jnp.int32)
    cnt_ref[...] = csum[CHUNK - 1:CHUNK, :].reshape(1, 1, E)


def _gate_call(x_f, W_gate, b_gate):
    return pl.pallas_call(
        _gate_body,
        grid=(NW,),
        in_specs=[
            pl.BlockSpec((CHUNK, C), lambda i: (i, 0)),
            pl.BlockSpec((C, E), lambda i: (0, 0)),
            pl.BlockSpec((1, E), lambda i: (0, 0)),
        ],
        out_specs=[
            pl.BlockSpec((CHUNK, E), lambda i: (i, 0)),
            pl.BlockSpec((CHUNK, 1), lambda i: (i, 0)),
            pl.BlockSpec((1, 1, E), lambda i: (i, 0, 0)),
        ],
        out_shape=[
            jax.ShapeDtypeStruct((N, E), jnp.float32),
            jax.ShapeDtypeStruct((N, 1), jnp.int32),
            jax.ShapeDtypeStruct((NW, 1, E), jnp.float32),
        ],
    )(x_f, W_gate, b_gate.reshape(1, E))


def _pos_body(gate_ref, rank_ref, base_ref, pos_ref):
    base8 = base_ref[...].reshape(1, E)
    sel = jnp.sum(gate_ref[...] * base8, axis=-1, keepdims=True)
    pos = rank_ref[...].astype(jnp.float32) + sel
    pos_ref[...] = pos.astype(jnp.int32)


def _pos_call(gate, rank, base_f):
    return pl.pallas_call(
        _pos_body,
        grid=(NW,),
        in_specs=[
            pl.BlockSpec((CHUNK, E), lambda i: (i, 0)),
            pl.BlockSpec((CHUNK, 1), lambda i: (i, 0)),
            pl.BlockSpec((1, 1, E), lambda i: (i, 0, 0)),
        ],
        out_specs=pl.BlockSpec((CHUNK, 1), lambda i: (i, 0)),
        out_shape=jax.ShapeDtypeStruct((N, 1), jnp.int32),
    )(gate, rank, base_f)


def _route_body(pos_hbm, x_hbm, xs_hbm, pos_v, buf_a, buf_b, sem_a, sem_b):
    wid = lax.axis_index("s") * 2 + lax.axis_index("c")
    tok0 = wid * CHUNK
    pltpu.sync_copy(pos_hbm.at[wid], pos_v)
    bufs = (buf_a, buf_b)
    sems = (sem_a, sem_b)
    copies = []
    for j in range(CHUNK // 16):
        bf = bufs[j % 2]
        if j >= 2:
            copies[j - 2].wait()
        pltpu.sync_copy(x_hbm.at[pl.ds(tok0 + j * 16, 16)], bf)
        copies.append(pltpu.async_copy(bf, xs_hbm.at[pos_v.at[j]],
                                       sems[j % 2]))
    copies[-2].wait()
    copies[-1].wait()


def _route_call(pos, x_f):
    mesh = plsc.VectorSubcoreMesh(core_axis_name="c", subcore_axis_name="s")
    return pl.kernel(
        _route_body,
        out_type=jax.ShapeDtypeStruct((N, C), jnp.float32),
        mesh=mesh,
        scratch_types=[
            pltpu.VMEM((CHUNK // 16, 16), jnp.int32),
            pltpu.VMEM((16, C), jnp.float32),
            pltpu.VMEM((16, C), jnp.float32),
            pltpu.SemaphoreType.DMA,
            pltpu.SemaphoreType.DMA,
        ],
    )(pos, x_f)


def _gmm_body(meta_ref, xs_ref, wu_ref, bu_ref, wd_ref, bd_ref, out_ref):
    w = pl.program_id(0)
    t = meta_ref[0, w]
    lo = meta_ref[2, w]
    hi = meta_ref[3, w]
    first = meta_ref[4, w]

    @pl.when(hi > lo)
    def _():
        rows = t * BM + lax.broadcasted_iota(jnp.int32, (BM, 1), 0)
        mask = (rows >= lo) & (rows < hi)
        h = jnp.dot(xs_ref[...], wu_ref[...],
                    preferred_element_type=jnp.float32) + bu_ref[...]
        h = jnp.maximum(h, 0.0)
        contrib = jnp.dot(h, wd_ref[...], preferred_element_type=jnp.float32)
        contrib = jnp.where(mask, contrib, 0.0)

        @pl.when(first == 1)
        def _():
            out_ref[...] = contrib + bd_ref[...]

        @pl.when(first == 0)
        def _():
            out_ref[...] = out_ref[...] + contrib


def _gmm_call(meta, xs, W_up, b_up, W_down, b_down):
    grid_spec = pltpu.PrefetchScalarGridSpec(
        num_scalar_prefetch=1,
        grid=(NSLOT,),
        in_specs=[
            pl.BlockSpec((BM, C), lambda i, m: (m[0, i], 0)),
            pl.BlockSpec((C, TS), lambda i, m: (0, m[1, i])),
            pl.BlockSpec((1, TS), lambda i, m: (0, m[1, i])),
            pl.BlockSpec((TS, C), lambda i, m: (m[1, i], 0)),
            pl.BlockSpec((1, C), lambda i, m: (0, 0)),
        ],
        out_specs=pl.BlockSpec((BM, C), lambda i, m: (m[0, i], 0)),
    )
    return pl.pallas_call(
        _gmm_body,
        grid_spec=grid_spec,
        out_shape=jax.ShapeDtypeStruct((N, C), jnp.float32),
    )(meta, xs, W_up, b_up.reshape(1, F), W_down, b_down.reshape(1, C))


def _unsort_body(pos_hbm, outs_hbm, out_hbm, pos_v, buf_a, buf_b,
                 sem_a, sem_b):
    wid = lax.axis_index("s") * 2 + lax.axis_index("c")
    tok0 = wid * CHUNK
    pltpu.sync_copy(pos_hbm.at[wid], pos_v)
    bufs = (buf_a, buf_b)
    sems = (sem_a, sem_b)
    nj = CHUNK // 16
    d = [None] * nj
    d[0] = pltpu.async_copy(outs_hbm.at[pos_v.at[0]], bufs[0], sems[0])
    for j in range(nj):
        d[j].wait()
        if j + 1 < nj:
            d[j + 1] = pltpu.async_copy(outs_hbm.at[pos_v.at[j + 1]],
                                        bufs[(j + 1) % 2], sems[(j + 1) % 2])
        pltpu.sync_copy(bufs[j % 2], out_hbm.at[pl.ds(tok0 + j * 16, 16)])


def _unsort_call(pos, out_s):
    mesh = plsc.VectorSubcoreMesh(core_axis_name="c", subcore_axis_name="s")
    return pl.kernel(
        _unsort_body,
        out_type=jax.ShapeDtypeStruct((N, C), jnp.float32),
        mesh=mesh,
        scratch_types=[
            pltpu.VMEM((CHUNK // 16, 16), jnp.int32),
            pltpu.VMEM((16, C), jnp.float32),
            pltpu.VMEM((16, C), jnp.float32),
            pltpu.SemaphoreType.DMA,
            pltpu.SemaphoreType.DMA,
        ],
    )(pos, out_s)


def _routing_metadata(cnts):
    tot = jnp.sum(cnts, axis=0)
    expert_start = jnp.cumsum(tot) - tot
    expert_end = expert_start + tot
    chunk_excl = jnp.cumsum(cnts, axis=0) - cnts
    base = expert_start[None, :] + chunk_excl
    base_f = base.astype(jnp.float32).reshape(NW, 1, E)

    t0 = expert_start // BM
    t1 = (expert_end - 1) // BM
    n_e = jnp.where(tot > 0, t1 - t0 + 1, 0)
    cum = jnp.cumsum(n_e)
    ni = cum[E - 1]
    ofs = cum - n_e
    s = jnp.arange(NSLOT, dtype=jnp.int32)
    e_s = jnp.sum((s[:, None] >= cum[None, :]).astype(jnp.int32), axis=1)
    valid = s < ni
    e_c = jnp.clip(e_s, 0, E - 1)
    t_sv = t0[e_c] + (s - ofs[e_c])
    lo_sv = jnp.maximum(expert_start[e_c], t_sv * BM)
    hi_sv = jnp.minimum(expert_end[e_c], (t_sv + 1) * BM)
    t_last = jnp.take(t_sv, ni - 1)
    e_last = jnp.take(e_c, ni - 1)
    t_f = jnp.where(valid, t_sv, t_last)
    e_f = jnp.where(valid, e_c, e_last)
    lo_f = jnp.where(valid, lo_sv, 0)
    hi_f = jnp.where(valid, hi_sv, 0)
    prev_t = jnp.concatenate([jnp.full((1,), -1, jnp.int32), t_f[:-1]])
    first_f = (valid & (t_f != prev_t)).astype(jnp.int32)
    meta = jnp.stack([t_f, e_f, lo_f, hi_f, first_f]).astype(jnp.int32)
    return base_f, meta


def kernel(x, W_gate, b_gate, W_up, b_up, W_down, b_down):
    B, T, Cx = x.shape
    assert (B * T, Cx, W_gate.shape[1], W_up.shape[1]) == (N, C, E, F)
    x_f = x.reshape(N, C)

    gate, rank, cnts = _gate_call(x_f, W_gate, b_gate)
    cnts_i = cnts.reshape(NW, E).astype(jnp.int32)
    base_f, meta = _routing_metadata(cnts_i)

    pos = _pos_call(gate, rank, base_f).reshape(NW, CHUNK // 16, 16)
    xs = _route_call(pos, x_f)
    out_s = _gmm_call(meta, xs, W_up, b_up, W_down, b_down)
    out_f = _unsort_call(pos, out_s)
    return out_f.reshape(B, T, C), gate.reshape(B, T, E)


# speedup vs baseline: 1.6083x; 1.6083x over previous
import functools
import jax
import jax.numpy as jnp
from jax import lax
from jax.experimental import pallas as pl


def _ffn_body(x_ref, wg_ref, bg_ref, wu_ref, bu_ref, wd_ref, bd_ref,
              out_ref, gate_ref, *, ts):
    xb = x_ref[...]
    logits = jnp.dot(xb, wg_ref[...],
                     preferred_element_type=jnp.float32) + bg_ref[...]
    lane = lax.broadcasted_iota(jnp.int32, logits.shape, 1)
    mx = jnp.max(logits, axis=-1, keepdims=True)
    num_e = logits.shape[-1]
    idx = jnp.min(jnp.where(logits == mx, lane, num_e), axis=-1,
                  keepdims=True)
    onehot = (lane == idx).astype(jnp.float32)
    gate_ref[...] = onehot
    xb16 = xb.astype(jnp.bfloat16)
    wu16 = wu_ref[...].astype(jnp.bfloat16)
    h = jnp.dot(xb16, wu16, preferred_element_type=jnp.float32) + bu_ref[...]
    tile_of_feat = lax.broadcasted_iota(jnp.int32, h.shape, 1) // ts
    h = jnp.where(tile_of_feat == idx, h, 0.0)
    h = jnp.maximum(h, 0.0)
    wd16 = wd_ref[...].astype(jnp.bfloat16)
    out_ref[...] = jnp.dot(h.astype(jnp.bfloat16), wd16,
                           preferred_element_type=jnp.float32) + bd_ref[...]


def kernel(x, W_gate, b_gate, W_up, b_up, W_down, b_down):
    B, T, C = x.shape
    N = B * T
    E = W_gate.shape[1]
    F = W_up.shape[1]
    TS = F // E
    x_f = x.reshape(N, C)
    BM = min(256, N)

    body = functools.partial(_ffn_body, ts=TS)
    out, gate = pl.pallas_call(
        body,
        grid=(N // BM,),
        in_specs=[
            pl.BlockSpec((BM, C), lambda i: (i, 0)),
            pl.BlockSpec((C, E), lambda i: (0, 0)),
            pl.BlockSpec((1, E), lambda i: (0, 0)),
            pl.BlockSpec((C, F), lambda i: (0, 0)),
            pl.BlockSpec((1, F), lambda i: (0, 0)),
            pl.BlockSpec((F, C), lambda i: (0, 0)),
            pl.BlockSpec((1, C), lambda i: (0, 0)),
        ],
        out_specs=[
            pl.BlockSpec((BM, C), lambda i: (i, 0)),
            pl.BlockSpec((BM, E), lambda i: (i, 0)),
        ],
        out_shape=[
            jax.ShapeDtypeStruct((N, C), jnp.float32),
            jax.ShapeDtypeStruct((N, E), jnp.float32),
        ],
    )(x_f, W_gate, b_gate.reshape(1, E), W_up, b_up.reshape(1, F),
      W_down, b_down.reshape(1, C))
    return out.reshape(B, T, C), gate.reshape(B, T, E)
